# Initial kernel scaffold; baseline (speedup 1.0000x reference)
import jax
import jax.numpy as jnp
from jax.experimental import pallas as pl


def kernel(prices, days_index, items_index):
    raise NotImplementedError("write your pallas kernel here")


# speedup vs baseline: 29.1353x; 29.1353x over previous
import functools

import jax
import jax.numpy as jnp
from jax import lax
from jax.experimental import pallas as pl
from jax.experimental.pallas import tpu as pltpu
from jax.experimental.pallas import tpu_sc as plsc

N_ITEMS = 30490
N_DAYS = 1969

_MEAN_ROWS = 256
_MEAN_PAD = 30720

_NW = 32
_LANES = 16


def _mean_body(p_ref, o_ref):
    o_ref[...] = jnp.mean(p_ref[...], axis=1)


def _row_means(prices):
    grid = _MEAN_PAD // _MEAN_ROWS
    return pl.pallas_call(
        _mean_body,
        grid=(grid,),
        in_specs=[pl.BlockSpec((_MEAN_ROWS, N_DAYS), lambda i: (i, 0))],
        out_specs=pl.BlockSpec((_MEAN_ROWS,), lambda i: (i,)),
        out_shape=jax.ShapeDtypeStruct((_MEAN_PAD,), jnp.float32),
    )(prices)


def _sc_body(per_w, chunk, prices_hbm, days_hbm, items_hbm, mean_hbm,
             op_hbm, om_hbm, or_hbm,
             mean_v, days_v, items_v, price_v, meang_v, rel_v, sem):
    wid = lax.axis_index("s") * 2 + lax.axis_index("c")
    base = wid * per_w
    pltpu.sync_copy(mean_hbm, mean_v)

    def do_chunk(c, carry):
        off = base + c * chunk
        pltpu.sync_copy(days_hbm.at[pl.ds(off, chunk)], days_v)
        pltpu.sync_copy(items_hbm.at[pl.ds(off, chunk)], items_v)

        def idx_loop(i, carry2):
            s = pl.ds(i * _LANES, _LANES)
            it = items_v[s]
            days_v[s] = it * N_DAYS + days_v[s]
            meang_v[s] = plsc.load_gather(mean_v, [it])
            return carry2

        lax.fori_loop(0, chunk // _LANES, idx_loop, 0)
        pltpu.async_copy(prices_hbm.at[days_v], price_v, sem).wait()

        def div_loop(i, carry2):
            s = pl.ds(i * _LANES, _LANES)
            rel_v[s] = price_v[s] / meang_v[s]
            return carry2

        lax.fori_loop(0, chunk // _LANES, div_loop, 0)
        pltpu.sync_copy(price_v, op_hbm.at[pl.ds(off, chunk)])
        pltpu.sync_copy(meang_v, om_hbm.at[pl.ds(off, chunk)])
        pltpu.sync_copy(rel_v, or_hbm.at[pl.ds(off, chunk)])
        return carry

    lax.fori_loop(0, per_w // chunk, do_chunk, 0)


@functools.partial(jax.jit, static_argnames=("bl",))
def _sc_gather(prices_flat, days_flat, items_flat, mean_pad, *, bl):
    per_w = bl // _NW
    chunk = 6400
    assert per_w % chunk == 0
    mesh = plsc.VectorSubcoreMesh(core_axis_name="c", subcore_axis_name="s")
    out = jax.ShapeDtypeStruct((bl,), jnp.float32)
    k = pl.kernel(
        functools.partial(_sc_body, per_w, chunk),
        out_type=(out, out, out),
        mesh=mesh,
        compiler_params=pltpu.CompilerParams(needs_layout_passes=False),
        scratch_types=[
            pltpu.VMEM((_MEAN_PAD,), jnp.float32),
            pltpu.VMEM((chunk,), jnp.int32),
            pltpu.VMEM((chunk,), jnp.int32),
            pltpu.VMEM((chunk,), jnp.float32),
            pltpu.VMEM((chunk,), jnp.float32),
            pltpu.VMEM((chunk,), jnp.float32),
            pltpu.SemaphoreType.DMA,
        ],
    )
    return k(prices_flat, days_flat, items_flat, mean_pad)


def kernel(prices, days_index, items_index):
    b, l = days_index.shape
    bl = b * l
    mean_pad = _row_means(prices)
    gp, gm, gr = _sc_gather(
        prices.reshape(-1),
        days_index.reshape(-1).astype(jnp.int32),
        items_index.reshape(-1).astype(jnp.int32),
        mean_pad,
        bl=bl,
    )
    return gp.reshape(b, l), gm.reshape(b, l), gr.reshape(b, l)


# speedup vs baseline: 34.9342x; 1.1990x over previous
import functools

import jax
import jax.numpy as jnp
from jax import lax
from jax.experimental import pallas as pl
from jax.experimental.pallas import tpu as pltpu
from jax.experimental.pallas import tpu_sc as plsc

N_ITEMS = 30490
N_DAYS = 1969

_RB = 128
_GRID = 239
_ROWS_PAD = _RB * _GRID
_DPAD = 2048
_TBLK = _RB * _DPAD
_TSIZE = _ROWS_PAD * _DPAD

_NW = 32
_LANES = 16
_CHUNK = 6400


def _tc_body(p_ref, mean_ref, tbl_ref):
    x = p_ref[...]
    off = pl.multiple_of(pl.program_id(0) * _RB, _RB)
    mean_ref[pl.ds(off, _RB)] = jnp.mean(x, axis=1)
    xp = jnp.concatenate([x, jnp.zeros((_RB, _DPAD - N_DAYS), jnp.float32)],
                         axis=1)
    y = xp.reshape(_RB // 8, 8, _DPAD // 128, 128).transpose(0, 2, 1, 3)
    tbl_ref[...] = y.reshape(_TBLK)


def _mean_and_flat(prices):
    return pl.pallas_call(
        _tc_body,
        grid=(_GRID,),
        in_specs=[pl.BlockSpec((_RB, N_DAYS), lambda i: (i, 0))],
        out_specs=[
            pl.BlockSpec((_ROWS_PAD,), lambda i: (0,)),
            pl.BlockSpec((_TBLK,), lambda i: (i,)),
        ],
        out_shape=[
            jax.ShapeDtypeStruct((_ROWS_PAD,), jnp.float32),
            jax.ShapeDtypeStruct((_TSIZE,), jnp.float32),
        ],
    )(prices)


def _sc_body(per_w, tbl_hbm, days_hbm, items_hbm, mean_hbm,
             op_hbm, om_hbm, or_hbm,
             mean_v, days_v, items_v, price_v, meang_v, rel_v, sem):
    wid = lax.axis_index("s") * 2 + lax.axis_index("c")
    base = wid * per_w
    pltpu.sync_copy(mean_hbm, mean_v)

    def do_chunk(c, carry):
        off = base + c * _CHUNK
        pltpu.sync_copy(days_hbm.at[pl.ds(off, _CHUNK)], days_v)
        pltpu.sync_copy(items_hbm.at[pl.ds(off, _CHUNK)], items_v)

        def idx_loop(i, carry2):
            s = pl.ds(i * _LANES, _LANES)
            it = items_v[s]
            dy = days_v[s]
            days_v[s] = (
                ((it >> 3) << 14) + ((dy >> 7) << 10)
                + ((it & 7) << 7) + (dy & 127)
            )
            meang_v[s] = plsc.load_gather(mean_v, [it])
            return carry2

        lax.fori_loop(0, _CHUNK // _LANES, idx_loop, 0)
        pltpu.async_copy(tbl_hbm.at[days_v], price_v, sem).wait()

        def div_loop(i, carry2):
            s = pl.ds(i * _LANES, _LANES)
            rel_v[s] = price_v[s] / meang_v[s]
            return carry2

        lax.fori_loop(0, _CHUNK // _LANES, div_loop, 0)
        pltpu.sync_copy(price_v, op_hbm.at[pl.ds(off, _CHUNK)])
        pltpu.sync_copy(meang_v, om_hbm.at[pl.ds(off, _CHUNK)])
        pltpu.sync_copy(rel_v, or_hbm.at[pl.ds(off, _CHUNK)])
        return carry

    lax.fori_loop(0, per_w // _CHUNK, do_chunk, 0)


@functools.partial(jax.jit, static_argnames=("bl",))
def _sc_gather(tbl_flat, days_flat, items_flat, mean_pad, *, bl):
    per_w = bl // _NW
    assert per_w % _CHUNK == 0
    mesh = plsc.VectorSubcoreMesh(core_axis_name="c", subcore_axis_name="s")
    out = jax.ShapeDtypeStruct((bl,), jnp.float32)
    k = pl.kernel(
        functools.partial(_sc_body, per_w),
        out_type=(out, out, out),
        mesh=mesh,
        compiler_params=pltpu.CompilerParams(needs_layout_passes=False),
        scratch_types=[
            pltpu.VMEM((_ROWS_PAD,), jnp.float32),
            pltpu.VMEM((_CHUNK,), jnp.int32),
            pltpu.VMEM((_CHUNK,), jnp.int32),
            pltpu.VMEM((_CHUNK,), jnp.float32),
            pltpu.VMEM((_CHUNK,), jnp.float32),
            pltpu.VMEM((_CHUNK,), jnp.float32),
            pltpu.SemaphoreType.DMA,
        ],
    )
    return k(tbl_flat, days_flat, items_flat, mean_pad)


def kernel(prices, days_index, items_index):
    b, l = days_index.shape
    bl = b * l
    mean_pad, tbl_flat = _mean_and_flat(prices)
    gp, gm, gr = _sc_gather(
        tbl_flat,
        days_index.reshape(-1).astype(jnp.int32),
        items_index.reshape(-1).astype(jnp.int32),
        mean_pad,
        bl=bl,
    )
    return gp.reshape(b, l), gm.reshape(b, l), gr.reshape(b, l)


# speedup vs baseline: 47.4496x; 1.3583x over previous
import functools

import jax
import jax.numpy as jnp
from jax import lax
from jax.experimental import pallas as pl
from jax.experimental.pallas import tpu as pltpu
from jax.experimental.pallas import tpu_sc as plsc

N_ITEMS = 30490
N_DAYS = 1969

_CB = 128
_GRID = 239
_ITEMS_PAD = _CB * _GRID
_DPAD = 2048
_TBLK = _DPAD * _CB
_TSIZE = _GRID * _TBLK

_NW = 32
_LANES = 16
_CHUNK = 6400


def _tc_body(pt_ref, mean_ref, tbl_ref):
    x = pt_ref[...]
    mean_ref[...] = jnp.mean(x, axis=0)
    xp = jnp.concatenate(
        [x, jnp.zeros((_DPAD - N_DAYS, _CB), jnp.float32)], axis=0)
    tbl_ref[...] = xp.reshape(_TBLK)


def _mean_and_flat(prices_t):
    return pl.pallas_call(
        _tc_body,
        grid=(_GRID,),
        in_specs=[pl.BlockSpec((N_DAYS, _CB), lambda i: (0, i))],
        out_specs=[
            pl.BlockSpec((_CB,), lambda i: (i,)),
            pl.BlockSpec((_TBLK,), lambda i: (i,)),
        ],
        out_shape=[
            jax.ShapeDtypeStruct((_ITEMS_PAD,), jnp.float32),
            jax.ShapeDtypeStruct((_TSIZE,), jnp.float32),
        ],
    )(prices_t)


def _sc_body(per_w, tbl_hbm, days_hbm, items_hbm, mean_hbm,
             op_hbm, om_hbm, or_hbm,
             mean_v, days_v, items_v, price_v, meang_v, rel_v, sem):
    wid = lax.axis_index("s") * 2 + lax.axis_index("c")
    base = wid * per_w
    pltpu.sync_copy(mean_hbm, mean_v)

    def do_chunk(c, carry):
        off = base + c * _CHUNK
        pltpu.sync_copy(days_hbm.at[pl.ds(off, _CHUNK)], days_v)
        pltpu.sync_copy(items_hbm.at[pl.ds(off, _CHUNK)], items_v)

        def idx_loop(i, carry2):
            s = pl.ds(i * _LANES, _LANES)
            it = items_v[s]
            dy = days_v[s]
            days_v[s] = (
                ((it >> 7) << 18) + ((dy >> 3) << 10)
                + ((dy & 7) << 7) + (it & 127)
            )
            meang_v[s] = plsc.load_gather(mean_v, [it])
            return carry2

        lax.fori_loop(0, _CHUNK // _LANES, idx_loop, 0)
        pltpu.async_copy(tbl_hbm.at[days_v], price_v, sem).wait()

        def div_loop(i, carry2):
            s = pl.ds(i * _LANES, _LANES)
            rel_v[s] = price_v[s] / meang_v[s]
            return carry2

        lax.fori_loop(0, _CHUNK // _LANES, div_loop, 0)
        pltpu.sync_copy(price_v, op_hbm.at[pl.ds(off, _CHUNK)])
        pltpu.sync_copy(meang_v, om_hbm.at[pl.ds(off, _CHUNK)])
        pltpu.sync_copy(rel_v, or_hbm.at[pl.ds(off, _CHUNK)])
        return carry

    lax.fori_loop(0, per_w // _CHUNK, do_chunk, 0)


@functools.partial(jax.jit, static_argnames=("bl",))
def _sc_gather(tbl_flat, days_flat, items_flat, mean_pad, *, bl):
    per_w = bl // _NW
    assert per_w % _CHUNK == 0
    mesh = plsc.VectorSubcoreMesh(core_axis_name="c", subcore_axis_name="s")
    out = jax.ShapeDtypeStruct((bl,), jnp.float32)
    k = pl.kernel(
        functools.partial(_sc_body, per_w),
        out_type=(out, out, out),
        mesh=mesh,
        compiler_params=pltpu.CompilerParams(needs_layout_passes=False),
        scratch_types=[
            pltpu.VMEM((_ITEMS_PAD,), jnp.float32),
            pltpu.VMEM((_CHUNK,), jnp.int32),
            pltpu.VMEM((_CHUNK,), jnp.int32),
            pltpu.VMEM((_CHUNK,), jnp.float32),
            pltpu.VMEM((_CHUNK,), jnp.float32),
            pltpu.VMEM((_CHUNK,), jnp.float32),
            pltpu.SemaphoreType.DMA,
        ],
    )
    return k(tbl_flat, days_flat, items_flat, mean_pad)


def kernel(prices, days_index, items_index):
    b, l = days_index.shape
    bl = b * l
    mean_pad, tbl_flat = _mean_and_flat(prices.T)
    gp, gm, gr = _sc_gather(
        tbl_flat,
        days_index.reshape(-1).astype(jnp.int32),
        items_index.reshape(-1).astype(jnp.int32),
        mean_pad,
        bl=bl,
    )
    return gp.reshape(b, l), gm.reshape(b, l), gr.reshape(b, l)


# speedup vs baseline: 54.2765x; 1.1439x over previous
import functools

import jax
import jax.numpy as jnp
from jax import lax
from jax.experimental import pallas as pl
from jax.experimental.pallas import tpu as pltpu
from jax.experimental.pallas import tpu_sc as plsc

N_ITEMS = 30490
N_DAYS = 1969

_CB = 128
_GRID = 239
_ITEMS_PAD = _CB * _GRID
_DPAD = 2048
_TBLK = _DPAD * _CB
_TSIZE = _GRID * _TBLK

_NW = 32
_LANES = 16
_CHUNK = 6400


def _tc_body(pt_ref, mean_ref, tbl_ref):
    x = pt_ref[...]
    mean_ref[...] = jnp.mean(x, axis=0)
    xp = jnp.concatenate(
        [x, jnp.zeros((_DPAD - N_DAYS, _CB), jnp.float32)], axis=0)
    tbl_ref[...] = xp.reshape(_TBLK)


def _mean_and_flat(prices_t):
    return pl.pallas_call(
        _tc_body,
        grid=(_GRID,),
        in_specs=[pl.BlockSpec((N_DAYS, _CB), lambda i: (0, i))],
        out_specs=[
            pl.BlockSpec((_CB,), lambda i: (i,)),
            pl.BlockSpec((_TBLK,), lambda i: (i,)),
        ],
        out_shape=[
            jax.ShapeDtypeStruct((_ITEMS_PAD,), jnp.float32),
            jax.ShapeDtypeStruct((_TSIZE,), jnp.float32),
        ],
    )(prices_t)


def _sc_body(per_w, tbl_hbm, days_hbm, items_hbm, mean_hbm,
             op_hbm, om_hbm, or_hbm, mean_v,
             days_a, items_a, price_a, meang_a, rel_a, sem_a,
             days_b, items_b, price_b, meang_b, rel_b, sem_b):
    wid = lax.axis_index("s") * 2 + lax.axis_index("c")
    base = wid * per_w
    pltpu.sync_copy(mean_hbm, mean_v)
    n_chunks = per_w // _CHUNK
    bufs = (
        (days_a, items_a, price_a, meang_a, rel_a, sem_a),
        (days_b, items_b, price_b, meang_b, rel_b, sem_b),
    )

    def stage(c, buf):
        days_v, items_v, price_v, meang_v, _, sem = bufs[buf]
        off = base + c * _CHUNK
        pltpu.sync_copy(days_hbm.at[pl.ds(off, _CHUNK)], days_v)
        pltpu.sync_copy(items_hbm.at[pl.ds(off, _CHUNK)], items_v)

        def idx_loop(i, carry):
            s = pl.ds(i * _LANES, _LANES)
            it = items_v[s]
            dy = days_v[s]
            days_v[s] = (
                ((it >> 7) << 18) + ((dy >> 3) << 10)
                + ((dy & 7) << 7) + (it & 127)
            )
            meang_v[s] = plsc.load_gather(mean_v, [it])
            return carry

        lax.fori_loop(0, _CHUNK // _LANES, idx_loop, 0)
        pltpu.async_copy(tbl_hbm.at[days_v], price_v, sem)

    def drain(c, buf):
        days_v, items_v, price_v, meang_v, rel_v, sem = bufs[buf]
        off = base + c * _CHUNK
        pltpu.make_async_copy(tbl_hbm.at[days_v], price_v, sem).wait()

        def div_loop(i, carry):
            s = pl.ds(i * _LANES, _LANES)
            rel_v[s] = price_v[s] / meang_v[s]
            return carry

        lax.fori_loop(0, _CHUNK // _LANES, div_loop, 0)
        pltpu.sync_copy(price_v, op_hbm.at[pl.ds(off, _CHUNK)])
        pltpu.sync_copy(meang_v, om_hbm.at[pl.ds(off, _CHUNK)])
        pltpu.sync_copy(rel_v, or_hbm.at[pl.ds(off, _CHUNK)])

    stage(0, 0)
    stage(1, 1)

    def step(g, carry):
        c = g * 2
        drain(c, 0)
        stage(c + 2, 0)
        drain(c + 1, 1)
        stage(c + 3, 1)
        return carry

    lax.fori_loop(0, n_chunks // 2 - 1, step, 0)
    drain(n_chunks - 2, 0)
    drain(n_chunks - 1, 1)


@functools.partial(jax.jit, static_argnames=("bl",))
def _sc_gather(tbl_flat, days_flat, items_flat, mean_pad, *, bl):
    per_w = bl // _NW
    assert per_w % _CHUNK == 0
    mesh = plsc.VectorSubcoreMesh(core_axis_name="c", subcore_axis_name="s")
    out = jax.ShapeDtypeStruct((bl,), jnp.float32)
    k = pl.kernel(
        functools.partial(_sc_body, per_w),
        out_type=(out, out, out),
        mesh=mesh,
        compiler_params=pltpu.CompilerParams(needs_layout_passes=False),
        scratch_types=[
            pltpu.VMEM((_ITEMS_PAD,), jnp.float32),
        ] + 2 * [
            pltpu.VMEM((_CHUNK,), jnp.int32),
            pltpu.VMEM((_CHUNK,), jnp.int32),
            pltpu.VMEM((_CHUNK,), jnp.float32),
            pltpu.VMEM((_CHUNK,), jnp.float32),
            pltpu.VMEM((_CHUNK,), jnp.float32),
            pltpu.SemaphoreType.DMA,
        ],
    )
    return k(tbl_flat, days_flat, items_flat, mean_pad)


def kernel(prices, days_index, items_index):
    b, l = days_index.shape
    bl = b * l
    mean_pad, tbl_flat = _mean_and_flat(prices.T)
    gp, gm, gr = _sc_gather(
        tbl_flat,
        days_index.reshape(-1).astype(jnp.int32),
        items_index.reshape(-1).astype(jnp.int32),
        mean_pad,
        bl=bl,
    )
    return gp.reshape(b, l), gm.reshape(b, l), gr.reshape(b, l)
